# baseline (speedup 1.0000x reference)
import jax
import jax.numpy as jnp
from jax.experimental import pallas as pl


def _gat_layer_j(x, src, dst, W, a_src, a_dst, b, H, C, concat):
    n = x.shape[0]
    xw = (x @ W).reshape(n, H, C)
    e_src = (xw * a_src[None, :, :]).sum(-1)
    e_dst = (xw * a_dst[None, :, :]).sum(-1)
    e = jax.nn.leaky_relu(e_src[src] + e_dst[dst], negative_slope=0.2)
    emax = jax.ops.segment_max(e, dst, num_segments=n)
    emax = jnp.where(jnp.isfinite(emax), emax, 0.0)
    ee = jnp.exp(e - emax[dst])
    denom = jax.ops.segment_sum(ee, dst, num_segments=n)
    alpha = ee / (denom[dst] + 1e-16)
    out = jax.ops.segment_sum(xw[src] * alpha[:, :, None], dst, num_segments=n)
    if concat:
        out = out.reshape(n, H * C)
    else:
        out = out.mean(axis=1)
    return out + b


def _logsoftmax_kernel(x_ref, o_ref):
    x = x_ref[...]
    m = jnp.max(x, axis=-1, keepdims=True)
    s = jnp.log(jnp.sum(jnp.exp(x - m), axis=-1, keepdims=True))
    o_ref[...] = x - m - s


def kernel(x, edge_index, W1, as1, ad1, b1, W2, as2, ad2, b2, W3, as3, ad3, b3, W4, as4, ad4, b4, Wskip, bskip):
    n = x.shape[0]
    loops = jnp.arange(n, dtype=edge_index.dtype)
    ei = jnp.concatenate([edge_index, jnp.stack([loops, loops])], axis=1)
    src, dst = ei[0], ei[1]
    x1 = jax.nn.relu(_gat_layer_j(x, src, dst, W1, as1, ad1, b1, 16, 128, True))
    x2 = jax.nn.relu(_gat_layer_j(x1, src, dst, W2, as2, ad2, b2, 16, 64, True))
    x3 = jax.nn.relu(_gat_layer_j(x2, src, dst, W3, as3, ad3, b3, 16, 32, True))
    x4 = _gat_layer_j(x3, src, dst, W4, as4, ad4, b4, 1, 10, False)
    logits = x4 + x @ Wskip + bskip
    pad = jnp.zeros((n, 118), jnp.float32) - 1e30
    lp = jnp.concatenate([logits, pad], axis=1)
    out = pl.pallas_call(
        _logsoftmax_kernel,
        out_shape=jax.ShapeDtypeStruct((n, 128), jnp.float32),
        grid=(n // 400,),
        in_specs=[pl.BlockSpec((400, 128), lambda i: (i, 0))],
        out_specs=pl.BlockSpec((400, 128), lambda i: (i, 0)),
    )(lp)
    return out[:, :10]


# speedup vs baseline: 2.8542x; 2.8542x over previous
import functools

import jax
import jax.numpy as jnp
from jax import lax
from jax.experimental import pallas as pl
from jax.experimental.pallas import tpu as pltpu
from jax.experimental.pallas import tpu_sc as plsc

NREAL = 10000
NP = 10240
NWORK = 32
NLOC = 320
RS_W = 336
G = 16


def _mm_attn_body(x_ref, w_ref, as_ref, ad_ref, xw_ref, es_ref, ed_ref):
    xw = jnp.dot(x_ref[...], w_ref[...], preferred_element_type=jnp.float32)
    xw_ref[...] = xw
    es_ref[...] = jnp.dot(xw, as_ref[...], preferred_element_type=jnp.float32)
    ed_ref[...] = jnp.dot(xw, ad_ref[...], preferred_element_type=jnp.float32)


def _mm_attn(x, Wp, Aps, Apd):
    n, K = x.shape
    F = Wp.shape[1]
    BN = 256
    return pl.pallas_call(
        _mm_attn_body,
        grid=(n // BN,),
        in_specs=[
            pl.BlockSpec((BN, K), lambda i: (i, 0)),
            pl.BlockSpec((K, F), lambda i: (0, 0)),
            pl.BlockSpec((F, 128), lambda i: (0, 0)),
            pl.BlockSpec((F, 128), lambda i: (0, 0)),
        ],
        out_specs=[
            pl.BlockSpec((BN, F), lambda i: (i, 0)),
            pl.BlockSpec((BN, 128), lambda i: (i, 0)),
            pl.BlockSpec((BN, 128), lambda i: (i, 0)),
        ],
        out_shape=[
            jax.ShapeDtypeStruct((n, F), jnp.float32),
            jax.ShapeDtypeStruct((n, 128), jnp.float32),
            jax.ShapeDtypeStruct((n, 128), jnp.float32),
        ],
    )(x, Wp, Aps, Apd)


def _final_body(x_ref, w_ref, b_ref, x4_ref, o_ref):
    logits = jnp.dot(x_ref[...], w_ref[...], preferred_element_type=jnp.float32)
    logits = logits + b_ref[...] + x4_ref[...]
    lane = lax.broadcasted_iota(jnp.int32, logits.shape, 1)
    logits = jnp.where(lane < 10, logits, -1e30)
    m = jnp.max(logits, axis=-1, keepdims=True)
    s = jnp.log(jnp.sum(jnp.exp(logits - m), axis=-1, keepdims=True))
    o_ref[...] = logits - m - s


def _final(x, Wsp, bsp, x4):
    n = x.shape[0]
    BN = 256
    return pl.pallas_call(
        _final_body,
        grid=(n // BN,),
        in_specs=[
            pl.BlockSpec((BN, 128), lambda i: (i, 0)),
            pl.BlockSpec((128, 128), lambda i: (0, 0)),
            pl.BlockSpec((1, 128), lambda i: (0, 0)),
            pl.BlockSpec((BN, 128), lambda i: (i, 0)),
        ],
        out_specs=pl.BlockSpec((BN, 128), lambda i: (i, 0)),
        out_shape=jax.ShapeDtypeStruct((n, 128), jnp.float32),
    )(x, Wsp, bsp, x4)


def _sc_layer(xw, es, ed_flat, srcs_pad, rs_pad, bias, F, FO, relu):
    CV = F // 16
    CVB = 8 if CV % 8 == 0 else 1
    NB = CV // CVB
    mesh = plsc.VectorSubcoreMesh(core_axis_name="c", subcore_axis_name="s")

    @functools.partial(
        pl.kernel,
        mesh=mesh,
        out_type=jax.ShapeDtypeStruct((NP * FO,), jnp.float32),
        scratch_types=[
            pltpu.VMEM((RS_W,), jnp.int32),
            pltpu.VMEM((G,), jnp.int32),
            pltpu.VMEM((G, 128), jnp.float32),
            pltpu.VMEM((128,), jnp.float32),
            pltpu.VMEM((G, F), jnp.float32),
            pltpu.VMEM((CV * 16,), jnp.float32),
            pltpu.VMEM((FO,), jnp.float32),
            pltpu.VMEM((F,), jnp.float32),
            pltpu.SemaphoreType.DMA,
        ],
    )
    def body(xw_h, es_h, edf_h, srcs_h, rs_h, bias_h, out_h,
             rs_loc, idxb, esb, edvb, rowb, accb, outb, biasb, sem):
        c = lax.axis_index("c")
        s = lax.axis_index("s")
        wid = s * 2 + c
        v0 = wid * NLOC
        cnt = jnp.minimum(NLOC, NREAL - v0)
        pltpu.sync_copy(rs_h.at[pl.ds(pl.multiple_of(v0, 16), RS_W)], rs_loc)
        pltpu.sync_copy(bias_h, biasb)
        zero16 = jnp.zeros((16,), jnp.float32)
        for i in range(CV):
            accb[pl.ds(i * 16, 16)] = zero16
        for i in range(FO // 16):
            outb[pl.ds(i * 16, 16)] = zero16
        neg = jnp.full((16,), -1e30, jnp.float32)

        def node_body(nv, _):
            v = v0 + nv
            rsv = rs_loc[pl.ds(nv, 16)]
            a = rsv[0]
            b = rsv[1]
            deg = b - a
            base0 = pl.multiple_of((a // 16) * 16, 16)
            skip = a - base0
            tot = skip + deg
            nch = (tot + (G - 1)) // G
            pltpu.sync_copy(edf_h.at[pl.ds(pl.multiple_of(v * 128, 128), 128)],
                            edvb)
            edv = edvb[pl.ds(0, 16)]

            def cha(t, m):
                pltpu.sync_copy(
                    srcs_h.at[pl.ds(pl.multiple_of(base0 + t * G, 16), G)],
                    idxb)
                pltpu.async_copy(es_h.at[idxb], esb, sem).wait()
                for j in range(G):
                    pos = t * G + j
                    okf = jnp.where((pos >= skip) & (pos < tot), 1.0, 0.0)
                    okv = jnp.broadcast_to(okf, (16,))
                    e0 = esb[j, pl.ds(0, 16)] + edv
                    e = jnp.maximum(e0, 0.2 * e0)
                    m = jnp.maximum(m, okv * e + (1.0 - okv) * neg)
                return m

            m = lax.fori_loop(0, nch, cha, neg)

            def chb(t, d):
                pltpu.sync_copy(
                    srcs_h.at[pl.ds(pl.multiple_of(base0 + t * G, 16), G)],
                    idxb)
                pltpu.async_copy(es_h.at[idxb], esb, sem).wait()
                pltpu.async_copy(xw_h.at[idxb], rowb, sem).wait()
                for j in range(G):
                    pos = t * G + j
                    okf = jnp.where((pos >= skip) & (pos < tot), 1.0, 0.0)
                    okv = jnp.broadcast_to(okf, (16,))
                    e0 = esb[j, pl.ds(0, 16)] + edv
                    e = jnp.maximum(e0, 0.2 * e0)
                    p = okv * jnp.exp(e - m)
                    d = d + p
                    if NB > 1:
                        def accum(ib, _, p=p, j=j):
                            for u in range(CVB):
                                off = ib * (CVB * 16) + u * 16
                                plsc.addupdate(accb.at[pl.ds(off, 16)],
                                               p * rowb[j, pl.ds(off, 16)])
                            return 0
                        lax.fori_loop(0, NB, accum, 0)
                    else:
                        for u in range(CV):
                            plsc.addupdate(accb.at[pl.ds(u * 16, 16)],
                                           p * rowb[j, pl.ds(u * 16, 16)])
                return d

            d = lax.fori_loop(0, nch, chb, jnp.zeros((16,), jnp.float32))

            inv = 1.0 / d
            for u in range(CV):
                o = accb[pl.ds(u * 16, 16)] * inv + biasb[pl.ds(u * 16, 16)]
                if relu:
                    o = jnp.maximum(o, 0.0)
                outb[pl.ds(u * 16, 16)] = o
                accb[pl.ds(u * 16, 16)] = zero16
            pltpu.sync_copy(outb,
                            out_h.at[pl.ds(pl.multiple_of(v * FO, 16), FO)])
            return 0

        lax.fori_loop(0, cnt, node_body, 0)

    return body(xw, es, ed_flat, srcs_pad, rs_pad, bias)


def _perm_cols(W, H, C):
    D = W.shape[0]
    return W.reshape(D, H, C).transpose(0, 2, 1).reshape(D, H * C)


def _perm_rows(W, H, C):
    F = W.shape[1]
    return W.reshape(H, C, F).transpose(1, 0, 2).reshape(H * C, F)


def _attn_mat(a, H, C):
    A3 = a.T[:, :, None] * jnp.eye(H, dtype=jnp.float32)[None, :, :]
    A = A3.reshape(C * H, H)
    return jnp.pad(A, ((0, 0), (0, 128 - H)))


def _bias_hm(b, H, C):
    return b.reshape(H, C).T.reshape(H * C)


def kernel(x, edge_index, W1, as1, ad1, b1, W2, as2, ad2, b2, W3, as3, ad3,
           b3, W4, as4, ad4, b4, Wskip, bskip):
    n = x.shape[0]

    loops = jnp.arange(n, dtype=edge_index.dtype)
    src = jnp.concatenate([edge_index[0], loops])
    dst = jnp.concatenate([edge_index[1], loops])
    order = jnp.argsort(dst)
    dsts = dst[order]
    srcs = src[order].astype(jnp.int32)
    row_start = jnp.searchsorted(dsts, jnp.arange(n + 1, dtype=jnp.int32)
                                 ).astype(jnp.int32)
    e_tot = srcs.shape[0]
    epad = ((e_tot + 63) // 64) * 64 + 64
    srcs_pad = jnp.concatenate(
        [srcs, jnp.zeros((epad - e_tot,), jnp.int32)])
    rs_pad = jnp.concatenate(
        [row_start,
         jnp.full((NWORK * NLOC + RS_W - (n + 1),), e_tot, jnp.int32)])

    W1p = _perm_cols(W1, 16, 128)
    W2p = _perm_cols(_perm_rows(W2, 16, 128), 16, 64)
    W3p = _perm_cols(_perm_rows(W3, 16, 64), 16, 32)
    W4r = _perm_rows(W4, 16, 32)
    W4p = jnp.pad(W4r, ((0, 0), (0, 118)))
    A1s, A1d = _attn_mat(as1, 16, 128), _attn_mat(ad1, 16, 128)
    A2s, A2d = _attn_mat(as2, 16, 64), _attn_mat(ad2, 16, 64)
    A3s, A3d = _attn_mat(as3, 16, 32), _attn_mat(ad3, 16, 32)
    a4p = jnp.pad(as4[0], (0, 118))
    a4pd = jnp.pad(ad4[0], (0, 118))
    lane16 = (jnp.arange(128) < 16).astype(jnp.float32)
    A4s = a4p[:, None] * lane16[None, :]
    A4d = a4pd[:, None] * lane16[None, :]
    b1p = _bias_hm(b1, 16, 128)
    b2p = _bias_hm(b2, 16, 64)
    b3p = _bias_hm(b3, 16, 32)
    b4p = jnp.pad(b4, (0, 118))
    Wsp = jnp.pad(Wskip, ((0, 0), (0, 118)))
    bsp = jnp.pad(bskip, (0, 118)).reshape(1, 128)

    xp = jnp.pad(x, ((0, NP - n), (0, 0)))

    def layer(xin, Wp, As, Ad, bp, F, FO, relu):
        xw, es_b, ed_b = _mm_attn(xin, Wp, As, Ad)
        ed_flat = ed_b.reshape(-1)
        of = _sc_layer(xw, es_b, ed_flat, srcs_pad, rs_pad, bp, F, FO, relu)
        return of.reshape(NP, FO)

    x1 = layer(xp, W1p, A1s, A1d, b1p, 2048, 2048, True)
    x2 = layer(x1, W2p, A2s, A2d, b2p, 1024, 1024, True)
    x3 = layer(x2, W3p, A3s, A3d, b3p, 512, 512, True)
    x4 = layer(x3, W4p, A4s, A4d, b4p, 128, 128, False)

    out = _final(xp, Wsp, bsp, x4)
    return out[:n, :10]


# speedup vs baseline: 4.9551x; 1.7360x over previous
import functools

import jax
import jax.numpy as jnp
from jax import lax
from jax.experimental import pallas as pl
from jax.experimental.pallas import tpu as pltpu
from jax.experimental.pallas import tpu_sc as plsc

NREAL = 10000
NP = 10240
NWORK = 32
NLOC = 320
RS_W = 336


def _mm_attn_body(x_ref, w_ref, as_ref, ad_ref, xw_ref, es_ref, ed_ref):
    xw = jnp.dot(x_ref[...], w_ref[...], preferred_element_type=jnp.float32)
    xw_ref[...] = xw
    es_ref[...] = jnp.dot(xw, as_ref[...], preferred_element_type=jnp.float32)
    ed_ref[...] = jnp.dot(xw, ad_ref[...], preferred_element_type=jnp.float32)


def _mm_attn(x, Wp, Aps, Apd):
    n, K = x.shape
    F = Wp.shape[1]
    BN = 256
    return pl.pallas_call(
        _mm_attn_body,
        grid=(n // BN,),
        in_specs=[
            pl.BlockSpec((BN, K), lambda i: (i, 0)),
            pl.BlockSpec((K, F), lambda i: (0, 0)),
            pl.BlockSpec((F, 128), lambda i: (0, 0)),
            pl.BlockSpec((F, 128), lambda i: (0, 0)),
        ],
        out_specs=[
            pl.BlockSpec((BN, F), lambda i: (i, 0)),
            pl.BlockSpec((BN, 128), lambda i: (i, 0)),
            pl.BlockSpec((BN, 128), lambda i: (i, 0)),
        ],
        out_shape=[
            jax.ShapeDtypeStruct((n, F), jnp.float32),
            jax.ShapeDtypeStruct((n, 128), jnp.float32),
            jax.ShapeDtypeStruct((n, 128), jnp.float32),
        ],
    )(x, Wp, Aps, Apd)


def _final_body(x_ref, w_ref, b_ref, x4_ref, o_ref):
    logits = jnp.dot(x_ref[...], w_ref[...], preferred_element_type=jnp.float32)
    logits = logits + b_ref[...] + x4_ref[...]
    lane = lax.broadcasted_iota(jnp.int32, logits.shape, 1)
    logits = jnp.where(lane < 10, logits, -1e30)
    m = jnp.max(logits, axis=-1, keepdims=True)
    s = jnp.log(jnp.sum(jnp.exp(logits - m), axis=-1, keepdims=True))
    o_ref[...] = logits - m - s


def _final(x, Wsp, bsp, x4):
    n = x.shape[0]
    BN = 256
    return pl.pallas_call(
        _final_body,
        grid=(n // BN,),
        in_specs=[
            pl.BlockSpec((BN, 128), lambda i: (i, 0)),
            pl.BlockSpec((128, 128), lambda i: (0, 0)),
            pl.BlockSpec((1, 128), lambda i: (0, 0)),
            pl.BlockSpec((BN, 128), lambda i: (i, 0)),
        ],
        out_specs=pl.BlockSpec((BN, 128), lambda i: (i, 0)),
        out_shape=jax.ShapeDtypeStruct((n, 128), jnp.float32),
    )(x, Wsp, bsp, x4)


def _sc_layer(xw, es, ed16_flat, blk, rs_pad, bias, F, relu):
    CV = F // 16
    CVB = 8 if CV % 8 == 0 else 1
    NB = CV // CVB
    mesh = plsc.VectorSubcoreMesh(core_axis_name="c", subcore_axis_name="s")

    @functools.partial(
        pl.kernel,
        mesh=mesh,
        out_type=jax.ShapeDtypeStruct((NP * F,), jnp.float32),
        scratch_types=[
            pltpu.VMEM((RS_W,), jnp.int32),
            pltpu.VMEM((2, 16), jnp.int32),
            pltpu.VMEM((2, 16), jnp.int32),
            pltpu.VMEM((16, 128), jnp.float32),
            pltpu.VMEM((16, 128), jnp.float32),
            pltpu.VMEM((16, F), jnp.float32),
            pltpu.VMEM((16, F), jnp.float32),
            pltpu.VMEM((NLOC * 16,), jnp.float32),
            pltpu.VMEM((NLOC * 16,), jnp.float32),
            pltpu.VMEM((CV * 16,), jnp.float32),
            pltpu.VMEM((F,), jnp.float32),
            pltpu.VMEM((F,), jnp.float32),
            pltpu.SemaphoreType.DMA,
            pltpu.SemaphoreType.DMA,
            pltpu.SemaphoreType.DMA,
            pltpu.SemaphoreType.DMA,
        ],
    )
    def body(xw_h, es_h, edf_h, blk_h, rs_h, bias_h, out_h,
             rs_loc, prbA, prbB, esbA, esbB, rowbA, rowbB,
             ed_loc, m_loc, accb, outb, biasb, semA, semB, semPA, semPB):
        c = lax.axis_index("c")
        s = lax.axis_index("s")
        wid = s * 2 + c
        v0 = wid * NLOC
        cnt = jnp.minimum(NLOC, NREAL - v0)
        pltpu.sync_copy(rs_h.at[pl.ds(pl.multiple_of(v0, 16), RS_W)], rs_loc)
        pltpu.sync_copy(bias_h, biasb)
        pltpu.sync_copy(
            edf_h.at[pl.ds(pl.multiple_of(v0 * 16, 16), NLOC * 16)], ed_loc)
        zero16 = jnp.zeros((16,), jnp.float32)
        for i in range(CV):
            accb[pl.ds(i * 16, 16)] = zero16

        def zml(i, _):
            m_loc[pl.ds(i * 16, 16)] = zero16
            return 0

        lax.fori_loop(0, NLOC, zml, 0)

        a0 = rs_loc[pl.ds(0, 16)][0]
        b_end = rs_loc[pl.ds(cnt, 16)][0]
        base0 = (a0 // 16) * 16
        cbase = base0 // 16
        skip = a0 - base0
        tot = skip + (b_end - a0)
        nchw = (tot + 15) // 16
        niter = (nchw + 1) // 2
        NEG = jnp.full((16,), -1e30, jnp.float32)

        def start_prb(tg, prbX, semPX):
            pltpu.make_async_copy(
                blk_h.at[pl.ds(pl.multiple_of(tg * 2, 2), 2)],
                prbX, semPX).start()

        def wait_prb(tg, prbX, semPX):
            pltpu.make_async_copy(
                blk_h.at[pl.ds(pl.multiple_of(tg * 2, 2), 2)],
                prbX, semPX).wait()

        def issue_gathers(prbX, esbX, rowbX, semX, with_rows):
            pltpu.make_async_copy(es_h.at[prbX.at[0]], esbX, semX).start()
            if with_rows:
                pltpu.make_async_copy(xw_h.at[prbX.at[0]], rowbX, semX).start()

        def wait_gathers(prbX, esbX, rowbX, semX, with_rows):
            pltpu.make_async_copy(es_h.at[prbX.at[0]], esbX, semX).wait()
            if with_rows:
                pltpu.make_async_copy(xw_h.at[prbX.at[0]], rowbX, semX).wait()

        start_prb(cbase, prbA, semPA)
        wait_prb(cbase, prbA, semPA)
        issue_gathers(prbA, esbA, rowbA, semA, False)
        start_prb(cbase + 1, prbB, semPB)

        def chunk1(t, carry, dstv, esbX):
            mv, cur = carry
            for j in range(16):
                pos = t * 16 + j
                okb = (pos >= skip) & (pos < tot)
                okv = jnp.broadcast_to(jnp.where(okb, 1.0, 0.0), (16,))
                dstj = dstv[j]
                is_new = okb & (dstj != cur)

                @pl.when(is_new)
                def _(cur=cur, mv=mv):
                    m_loc[pl.ds((cur - v0) * 16, 16)] = mv

                selv = jnp.broadcast_to(jnp.where(is_new, 1.0, 0.0), (16,))
                li = jnp.minimum(jnp.maximum(dstj - v0, 0), NLOC - 1)
                edv = ed_loc[pl.ds(li * 16, 16)]
                e0 = esbX[j, pl.ds(0, 16)] + edv
                e = jnp.maximum(e0, 0.2 * e0)
                em = okv * e + (1.0 - okv) * NEG
                mv = selv * em + (1.0 - selv) * jnp.maximum(mv, em)
                cur = jnp.where(is_new, dstj, cur)
            return mv, cur

        def p1body(i, carry):
            t0 = 2 * i
            t1 = t0 + 1
            wait_prb(cbase + t1, prbB, semPB)
            issue_gathers(prbB, esbB, rowbB, semB, False)
            wait_gathers(prbA, esbA, rowbA, semA, False)
            dstvA = prbA[1]
            start_prb(cbase + t0 + 2, prbA, semPA)
            carry = chunk1(t0, carry, dstvA, esbA)
            wait_prb(cbase + t0 + 2, prbA, semPA)
            issue_gathers(prbA, esbA, rowbA, semA, False)
            wait_gathers(prbB, esbB, rowbB, semB, False)
            dstvB = prbB[1]
            start_prb(cbase + t1 + 2, prbB, semPB)
            carry = chunk1(t1, carry, dstvB, esbB)
            return carry

        mv, cur = lax.fori_loop(0, niter, p1body, (NEG, v0))
        m_loc[pl.ds((cur - v0) * 16, 16)] = mv
        wait_gathers(prbA, esbA, rowbA, semA, False)
        wait_prb(cbase, prbB, semPB)

        start_prb(cbase, prbA, semPA)
        wait_prb(cbase, prbA, semPA)
        issue_gathers(prbA, esbA, rowbA, semA, True)
        start_prb(cbase + 1, prbB, semPB)

        def flush(cur, d):
            inv = 1.0 / d

            def fb(ib, _):
                for u in range(CVB):
                    off = ib * (CVB * 16) + u * 16
                    o = accb[pl.ds(off, 16)] * inv + biasb[pl.ds(off, 16)]
                    if relu:
                        o = jnp.maximum(o, 0.0)
                    outb[pl.ds(off, 16)] = o
                    accb[pl.ds(off, 16)] = zero16
                return 0

            if NB > 1:
                lax.fori_loop(0, NB, fb, 0)
            else:
                fb(0, 0)
            pltpu.sync_copy(
                outb, out_h.at[pl.ds(pl.multiple_of(cur * F, 16), F)])

        def chunk2(t, carry, dstv, esbX, rowbX):
            d, cur = carry
            for j in range(16):
                pos = t * 16 + j
                okb = (pos >= skip) & (pos < tot)
                okv = jnp.broadcast_to(jnp.where(okb, 1.0, 0.0), (16,))
                dstj = dstv[j]
                is_new = okb & (dstj != cur)

                @pl.when(is_new)
                def _(cur=cur, d=d):
                    flush(cur, d)

                selv = jnp.broadcast_to(jnp.where(is_new, 1.0, 0.0), (16,))
                li = jnp.minimum(jnp.maximum(dstj - v0, 0), NLOC - 1)
                edv = ed_loc[pl.ds(li * 16, 16)]
                mvv = m_loc[pl.ds(li * 16, 16)]
                e0 = esbX[j, pl.ds(0, 16)] + edv
                e = jnp.maximum(e0, 0.2 * e0)
                p = okv * jnp.exp(jnp.minimum(e - mvv, 0.0))
                d = (1.0 - selv) * d + p
                cur = jnp.where(is_new, dstj, cur)

                def ab(ib, _, p=p, j=j):
                    for u in range(CVB):
                        off = ib * (CVB * 16) + u * 16
                        plsc.addupdate(accb.at[pl.ds(off, 16)],
                                       p * rowbX[j, pl.ds(off, 16)])
                    return 0

                if NB > 1:
                    lax.fori_loop(0, NB, ab, 0)
                else:
                    ab(0, 0)
            return d, cur

        def p2body(i, carry):
            t0 = 2 * i
            t1 = t0 + 1
            wait_prb(cbase + t1, prbB, semPB)
            issue_gathers(prbB, esbB, rowbB, semB, True)
            wait_gathers(prbA, esbA, rowbA, semA, True)
            dstvA = prbA[1]
            start_prb(cbase + t0 + 2, prbA, semPA)
            carry = chunk2(t0, carry, dstvA, esbA, rowbA)
            wait_prb(cbase + t0 + 2, prbA, semPA)
            issue_gathers(prbA, esbA, rowbA, semA, True)
            wait_gathers(prbB, esbB, rowbB, semB, True)
            dstvB = prbB[1]
            start_prb(cbase + t1 + 2, prbB, semPB)
            carry = chunk2(t1, carry, dstvB, esbB, rowbB)
            return carry

        d, cur = lax.fori_loop(0, niter, p2body,
                               (jnp.zeros((16,), jnp.float32), v0))
        flush(cur, d)
        wait_gathers(prbA, esbA, rowbA, semA, True)
        wait_prb(cbase, prbB, semPB)

    return body(xw, es, ed16_flat, blk, rs_pad, bias)


def _perm_cols(W, H, C):
    D = W.shape[0]
    return W.reshape(D, H, C).transpose(0, 2, 1).reshape(D, H * C)


def _perm_rows(W, H, C):
    F = W.shape[1]
    return W.reshape(H, C, F).transpose(1, 0, 2).reshape(H * C, F)


def _attn_mat(a, H, C):
    A3 = a.T[:, :, None] * jnp.eye(H, dtype=jnp.float32)[None, :, :]
    A = A3.reshape(C * H, H)
    return jnp.pad(A, ((0, 0), (0, 128 - H)))


def _bias_hm(b, H, C):
    return b.reshape(H, C).T.reshape(H * C)


def kernel(x, edge_index, W1, as1, ad1, b1, W2, as2, ad2, b2, W3, as3, ad3,
           b3, W4, as4, ad4, b4, Wskip, bskip):
    n = x.shape[0]

    loops = jnp.arange(n, dtype=edge_index.dtype)
    src = jnp.concatenate([edge_index[0], loops])
    dst = jnp.concatenate([edge_index[1], loops])
    order = jnp.argsort(dst)
    dsts = dst[order].astype(jnp.int32)
    srcs = src[order].astype(jnp.int32)
    row_start = jnp.searchsorted(dsts, jnp.arange(n + 1, dtype=jnp.int32)
                                 ).astype(jnp.int32)
    e_tot = srcs.shape[0]
    epad = ((e_tot + 15) // 16 + 8) * 16
    srcs_p = jnp.concatenate([srcs, jnp.zeros((epad - e_tot,), jnp.int32)])
    dsts_p = jnp.concatenate([dsts, jnp.zeros((epad - e_tot,), jnp.int32)])
    blk = jnp.concatenate([srcs_p.reshape(-1, 16), dsts_p.reshape(-1, 16)],
                          axis=1).reshape(-1, 16)
    rs_pad = jnp.concatenate(
        [row_start,
         jnp.full((NWORK * NLOC + RS_W - (n + 1),), e_tot, jnp.int32)])

    W1p = _perm_cols(W1, 16, 128)
    W2p = _perm_cols(_perm_rows(W2, 16, 128), 16, 64)
    W3p = _perm_cols(_perm_rows(W3, 16, 64), 16, 32)
    W4r = _perm_rows(W4, 16, 32)
    W4p = jnp.pad(W4r, ((0, 0), (0, 118)))
    A1s, A1d = _attn_mat(as1, 16, 128), _attn_mat(ad1, 16, 128)
    A2s, A2d = _attn_mat(as2, 16, 64), _attn_mat(ad2, 16, 64)
    A3s, A3d = _attn_mat(as3, 16, 32), _attn_mat(ad3, 16, 32)
    a4p = jnp.pad(as4[0], (0, 118))
    a4pd = jnp.pad(ad4[0], (0, 118))
    lane16 = (jnp.arange(128) < 16).astype(jnp.float32)
    A4s = a4p[:, None] * lane16[None, :]
    A4d = a4pd[:, None] * lane16[None, :]
    b1p = _bias_hm(b1, 16, 128)
    b2p = _bias_hm(b2, 16, 64)
    b3p = _bias_hm(b3, 16, 32)
    b4p = jnp.pad(b4, (0, 118))
    Wsp = jnp.pad(Wskip, ((0, 0), (0, 118)))
    bsp = jnp.pad(bskip, (0, 118)).reshape(1, 128)

    xp = jnp.pad(x, ((0, NP - n), (0, 0)))

    def layer(xin, Wp, As, Ad, bp, F, relu):
        xw, es_b, ed_b = _mm_attn(xin, Wp, As, Ad)
        ed16_flat = ed_b[:, :16].reshape(-1)
        of = _sc_layer(xw, es_b, ed16_flat, blk, rs_pad, bp, F, relu)
        return of.reshape(NP, F)

    x1 = layer(xp, W1p, A1s, A1d, b1p, 2048, True)
    x2 = layer(x1, W2p, A2s, A2d, b2p, 1024, True)
    x3 = layer(x2, W3p, A3s, A3d, b3p, 512, True)
    x4 = layer(x3, W4p, A4s, A4d, b4p, 128, False)

    out = _final(xp, Wsp, bsp, x4)
    return out[:n, :10]
